# baseline (speedup 1.0000x reference)
import functools

import jax
import jax.numpy as jnp
from jax import lax
from jax.experimental import pallas as pl
from jax.experimental.pallas import tpu as pltpu
from jax.experimental.pallas import tpu_sc as plsc

NC = 2
NS = 16
NW = NC * NS
CH = 128
GRP = 3
V = 8192


@functools.partial(jax.jit, static_argnums=(2, 3))
def _gather(weight, idx, nch, dim):
    bpw = nch * CH
    nchh = nch // 2
    ngrp = nchh // GRP
    ntail = nchh - ngrp * GRP
    rpt = V // NS

    mesh = plsc.VectorSubcoreMesh(core_axis_name="c", subcore_axis_name="s")

    @functools.partial(
        pl.kernel,
        mesh=mesh,
        out_type=jax.ShapeDtypeStruct((NW * bpw, dim), jnp.float32),
        scratch_types=[
            pltpu.VMEM((nchh, CH), jnp.int32),
            pltpu.VMEM((GRP * CH, dim), jnp.float32),
            pltpu.VMEM_SHARED((V, dim), jnp.float32),
            pltpu.SemaphoreType.DMA,
            pltpu.SemaphoreType.DMA,
        ],
    )
    def body(table_hbm, idx_hbm, out_hbm, idx_v, rows_v, tab_s, gsem, ssem):
        c = lax.axis_index("c")
        s = lax.axis_index("s")
        wid = s * NC + c
        base = wid * bpw

        src0 = s * rpt

        @pl.when(s == 0)
        def _first_stripe():
            pltpu.sync_copy(table_hbm.at[pl.ds(8, rpt - 8)], tab_s.at[pl.ds(7, rpt - 8)])
            pltpu.sync_copy(table_hbm.at[pl.ds(0, 8)], rows_v.at[pl.ds(0, 8)])
            pltpu.sync_copy(rows_v.at[pl.ds(1, 7)], tab_s.at[pl.ds(0, 7)])

        @pl.when(s != 0)
        def _stripe():
            pltpu.sync_copy(table_hbm.at[pl.ds(src0, rpt)], tab_s.at[pl.ds(src0 - 1, rpt)])

        @pl.when(s == NS - 1)
        def _last_row():
            pltpu.sync_copy(table_hbm.at[pl.ds(V, 1)], rows_v.at[pl.ds(0, 1)])
            pltpu.sync_copy(rows_v.at[pl.ds(0, 1)], tab_s.at[pl.ds(V - 1, 1)])

        plsc.subcore_barrier()

        def start_gather(j, b):
            pltpu.async_copy(tab_s.at[idx_v.at[j]], rows_v.at[pl.ds(b * CH, CH)], gsem)

        def start_scatter(jg, b):
            pltpu.async_copy(
                rows_v.at[pl.ds(b * CH, CH)], out_hbm.at[pl.ds(base + jg * CH, CH)], ssem
            )

        def wait_gather(b):
            pltpu.make_async_copy(
                tab_s.at[pl.ds(0, CH)], rows_v.at[pl.ds(b * CH, CH)], gsem
            ).wait()

        def wait_scatter(b):
            pltpu.make_async_copy(
                rows_v.at[pl.ds(b * CH, CH)], out_hbm.at[pl.ds(0, CH)], ssem
            ).wait()

        for h in range(2):
            pltpu.sync_copy(idx_hbm.at[wid].at[h], idx_v)

            for b in range(GRP):
                if h > 0:
                    wait_scatter(b)
                start_gather(b, b)
            for b in range(GRP):
                wait_gather(b)
                start_scatter(h * nchh + b, b)

            def group(g, carry):
                j0 = g * GRP
                for b in range(GRP):
                    wait_scatter(b)
                    start_gather(j0 + b, b)
                for b in range(GRP):
                    wait_gather(b)
                    start_scatter(h * nchh + j0 + b, b)
                return carry

            lax.fori_loop(1, ngrp, group, 0)

            for t in range(ntail):
                j = ngrp * GRP + t
                wait_scatter(t)
                start_gather(j, t)
            for t in range(ntail):
                wait_gather(t)
                start_scatter(h * nchh + ngrp * GRP + t, t)

        for b in range(GRP):
            wait_scatter(b)

    return body(weight, idx)


def kernel(input_positions, weight):
    bsz, slen = input_positions.shape
    dim = weight.shape[1]
    total = bsz * slen
    nch = total // (NW * CH)
    idx = input_positions.astype(jnp.int32).reshape(NW, 2, nch // 2, CH)
    out = _gather(weight, idx, nch, dim)
    return out.reshape(bsz, slen, dim)


# speedup vs baseline: 1.0008x; 1.0008x over previous
import functools

import jax
import jax.numpy as jnp
from jax import lax
from jax.experimental import pallas as pl
from jax.experimental.pallas import tpu as pltpu
from jax.experimental.pallas import tpu_sc as plsc

NC = 2
NS = 16
NW = NC * NS
CH = 80
GRP = 4
V = 8192


@functools.partial(jax.jit, static_argnums=(2, 3))
def _gather(weight, idx, nch, dim):
    bpw = nch * CH
    nchh = nch // 2
    ngrp = nchh // GRP
    rpt = V // NS

    mesh = plsc.VectorSubcoreMesh(core_axis_name="c", subcore_axis_name="s")

    @functools.partial(
        pl.kernel,
        mesh=mesh,
        out_type=jax.ShapeDtypeStruct((NW * bpw, dim), jnp.float32),
        scratch_types=[
            pltpu.VMEM((nchh, CH), jnp.int32),
            pltpu.VMEM((GRP * CH, dim), jnp.float32),
            pltpu.VMEM_SHARED((V, dim), jnp.float32),
            pltpu.SemaphoreType.DMA,
            pltpu.SemaphoreType.DMA,
        ],
    )
    def body(table_hbm, idx_hbm, out_hbm, idx_v, rows_v, tab_s, gsem, ssem):
        c = lax.axis_index("c")
        s = lax.axis_index("s")
        wid = s * NC + c
        base = wid * bpw

        src0 = s * rpt

        @pl.when(s == 0)
        def _first_stripe():
            pltpu.sync_copy(table_hbm.at[pl.ds(8, rpt - 8)], tab_s.at[pl.ds(7, rpt - 8)])
            pltpu.sync_copy(table_hbm.at[pl.ds(0, 8)], rows_v.at[pl.ds(0, 8)])
            pltpu.sync_copy(rows_v.at[pl.ds(1, 7)], tab_s.at[pl.ds(0, 7)])

        @pl.when(s != 0)
        def _stripe():
            pltpu.sync_copy(table_hbm.at[pl.ds(src0, rpt)], tab_s.at[pl.ds(src0 - 1, rpt)])

        @pl.when(s == NS - 1)
        def _last_row():
            pltpu.sync_copy(table_hbm.at[pl.ds(V, 1)], rows_v.at[pl.ds(0, 1)])
            pltpu.sync_copy(rows_v.at[pl.ds(0, 1)], tab_s.at[pl.ds(V - 1, 1)])

        plsc.subcore_barrier()

        def start_gather(j, b):
            pltpu.async_copy(tab_s.at[idx_v.at[j]], rows_v.at[pl.ds(b * CH, CH)], gsem)

        def start_scatter(jg, b):
            pltpu.async_copy(
                rows_v.at[pl.ds(b * CH, CH)], out_hbm.at[pl.ds(base + jg * CH, CH)], ssem
            )

        def wait_gather(b):
            pltpu.make_async_copy(
                tab_s.at[pl.ds(0, CH)], rows_v.at[pl.ds(b * CH, CH)], gsem
            ).wait()

        def wait_scatter(b):
            pltpu.make_async_copy(
                rows_v.at[pl.ds(b * CH, CH)], out_hbm.at[pl.ds(0, CH)], ssem
            ).wait()

        for h in range(2):
            pltpu.sync_copy(idx_hbm.at[wid].at[h], idx_v)

            for b in range(GRP):
                if h > 0:
                    wait_scatter(b)
                start_gather(b, b)
            for b in range(GRP):
                wait_gather(b)
                start_scatter(h * nchh + b, b)

            def group(g, carry):
                j0 = g * GRP
                for b in range(GRP):
                    wait_scatter(b)
                    start_gather(j0 + b, b)
                for b in range(GRP):
                    wait_gather(b)
                    start_scatter(h * nchh + j0 + b, b)
                return carry

            lax.fori_loop(1, ngrp, group, 0)

        for b in range(GRP):
            wait_scatter(b)

    return body(weight, idx)


def kernel(input_positions, weight):
    bsz, slen = input_positions.shape
    dim = weight.shape[1]
    total = bsz * slen
    nch = total // (NW * CH)
    idx = input_positions.astype(jnp.int32).reshape(NW, 2, nch // 2, CH)
    out = _gather(weight, idx, nch, dim)
    return out.reshape(bsz, slen, dim)
